# baseline (speedup 1.0000x reference)
import functools

import jax
import jax.numpy as jnp
from jax import lax
from jax.experimental import pallas as pl
from jax.experimental.pallas import tpu as pltpu
from jax.experimental.pallas import tpu_sc as plsc

NC = 2
NS = 16
NW = NC * NS
TW = 32
CH = 80


def _pack_bf16(lo_f32, hi_f32):
    lo = lax.bitcast_convert_type(lo_f32.astype(jnp.bfloat16), jnp.uint16)
    hi = lax.bitcast_convert_type(hi_f32.astype(jnp.bfloat16), jnp.uint16)
    return (lo.astype(jnp.int32)
            | lax.shift_left(hi.astype(jnp.int32), 16))


def _table_body(nf_ref, ma_ref, mb_ref, s1_ref, s2_ref, t1_ref, t2_ref):
    x = nf_ref[...]
    x2 = x * x
    sq = (jnp.dot(x, s1_ref[...], preferred_element_type=jnp.float32)
          + jnp.dot(x2, s2_ref[...], preferred_element_type=jnp.float32))
    pa = jnp.dot(x, ma_ref[...], preferred_element_type=jnp.float32)
    pb = jnp.dot(x, mb_ref[...], preferred_element_type=jnp.float32)
    t1_ref[...] = _pack_bf16(pa, sq)
    t2_ref[...] = _pack_bf16(pb, sq)


NBUF = 2


def _gather_body(per_w, n_ch, tsrc, tdst, ij, gx, ivm, jvm, *bufflat):
    ba = bufflat[0:NBUF]
    bb = bufflat[NBUF:2 * NBUF]
    bw = bufflat[2 * NBUF:3 * NBUF]
    sa = bufflat[3 * NBUF:4 * NBUF]
    sb = bufflat[4 * NBUF:5 * NBUF]
    sw = bufflat[5 * NBUF:6 * NBUF]
    wid = lax.axis_index("s") * NC + lax.axis_index("c")
    base = wid * per_w
    pltpu.sync_copy(ij.at[0, wid], ivm)
    pltpu.sync_copy(ij.at[1, wid], jvm)

    lanes = lax.iota(jnp.int32, 16)
    mhi = jnp.full((16,), -65536, jnp.int32)
    rows_sq = 16 + jnp.where(lanes == 1, 1, 0)
    msk_sq = lanes < 2

    def start(c, k):
        pltpu.async_copy(tsrc.at[ivm.at[c]], ba[k], sa[k])
        pltpu.async_copy(tdst.at[jvm.at[c]], bb[k], sb[k])

    def process(c, k):
        off = pl.multiple_of(c * CH, 8)
        pltpu.make_async_copy(tsrc.at[ivm.at[c]], ba[k], sa[k]).wait()
        pltpu.make_async_copy(tdst.at[jvm.at[c]], bb[k], sb[k]).wait()

        @pl.when(c >= NBUF)
        def _():
            pltpu.make_async_copy(
                bw[k], gx.at[pl.ds(0, 18), pl.ds(base + off, CH)], sw[k]).wait()

        for rr in range(CH):
            va = ba[k][rr, :]
            vb = bb[k][rr, :]
            cc = jnp.full((16,), rr, jnp.int32)
            pa = (plsc.bitcast(lax.shift_left(va, 16), jnp.float32)
                  + plsc.bitcast(lax.shift_left(vb, 16), jnp.float32))
            plsc.store_scatter(bw[k], [lanes, cc], pa)
            hs = (plsc.bitcast(lax.bitwise_and(va, mhi), jnp.float32)
                  + plsc.bitcast(lax.bitwise_and(vb, mhi), jnp.float32))
            plsc.store_scatter(bw[k], [rows_sq, cc], hs, mask=msk_sq)
        pltpu.async_copy(bw[k], gx.at[pl.ds(0, 18), pl.ds(base + off, CH)], sw[k])

    for k in range(NBUF - 1):
        start(k, k)

    def body(c2, carry):
        cb = NBUF * c2

        for k in range(NBUF):
            c = cb + k

            @pl.when(c + NBUF - 1 < n_ch)
            def _(c=c, k=k):
                start(c + NBUF - 1, (k + NBUF - 1) % NBUF)

            @pl.when(c < n_ch)
            def _(c=c, k=k):
                process(c, k)

        return carry

    lax.fori_loop(0, (n_ch + NBUF - 1) // NBUF, body, 0)

    for k in range(NBUF):
        lastc = (n_ch - 1 - k) // NBUF * NBUF + k
        if lastc >= 0:
            pltpu.make_async_copy(
                bw[k], gx.at[pl.ds(0, 18), pl.ds(base + lastc * CH, CH)],
                sw[k]).wait()


def _mlp_body(cat_dim, e_ref, gx_ref, cw_ref, uvb_ref, o_ref):
    et = e_ref[...]
    g1t = gx_ref[0:16, :]
    srow = gx_ref[16:17, :]
    qrow = gx_ref[17:18, :]
    ct = cw_ref[:, 0:16]
    w2t = cw_ref[:, 16:32]
    scale = 1.0 / cat_dim
    se = jnp.sum(et, axis=0, keepdims=True)
    qe = jnp.sum(et * et, axis=0, keepdims=True)
    mu = (srow + se) * scale
    var = (qrow + qe) * scale - mu * mu
    inv = lax.rsqrt(var + 1e-5)
    ect = jnp.dot(ct, et, preferred_element_type=jnp.float32)
    u = uvb_ref[:, 0:1]
    v = uvb_ref[:, 1:2]
    b2c = uvb_ref[:, 2:3]
    y1 = (g1t + ect) * inv - u * (mu * inv) + v
    y1 = jnp.where(y1 > 0, y1, 0.01 * y1)
    y2 = jnp.dot(w2t, y1, preferred_element_type=jnp.float32) + b2c
    o_ref[...] = et + y2


def kernel(h0, edge_index, edge_w, ln_g, ln_b, W1, b1, W2, b2, r, basis):
    N, D, _ = h0.shape
    E = edge_index.shape[1]
    ED = edge_w.shape[1]
    cat_dim = 2 * D + ED
    nf = jnp.squeeze(h0, axis=-1)

    w1g = W1 * ln_g[:, None]
    a_m = w1g[:D]
    b_m = w1g[D:2 * D]
    c_m = w1g[2 * D:]
    u = jnp.sum(w1g, axis=0)
    v = ln_b @ W1 + b1
    zcol = jnp.zeros((D, ED - 2), jnp.float32)
    one = jnp.ones((D, 1), jnp.float32)
    zero = jnp.zeros((D, 1), jnp.float32)
    s1m = jnp.concatenate([one, zero, zcol], axis=1)
    s2m = jnp.concatenate([zero, one, zcol], axis=1)

    cw = jnp.concatenate([c_m.T, W2.T], axis=1)
    uvb = jnp.concatenate([u[:, None], v[:, None], b2[:, None],
                           jnp.zeros((ED, 5), jnp.float32)], axis=1)

    nb = 5
    bn = N // nb
    t_src, t_dst = pl.pallas_call(
        _table_body,
        grid=(nb,),
        in_specs=[
            pl.BlockSpec((bn, D), lambda b: (b, 0)),
            pl.BlockSpec((D, ED), lambda b: (0, 0)),
            pl.BlockSpec((D, ED), lambda b: (0, 0)),
            pl.BlockSpec((D, ED), lambda b: (0, 0)),
            pl.BlockSpec((D, ED), lambda b: (0, 0)),
        ],
        out_specs=[
            pl.BlockSpec((bn, ED), lambda b: (b, 0)),
            pl.BlockSpec((bn, ED), lambda b: (b, 0)),
        ],
        out_shape=[
            jax.ShapeDtypeStruct((N, ED), jnp.int32),
            jax.ShapeDtypeStruct((N, ED), jnp.int32),
        ],
    )(nf, a_m, b_m, s1m, s2m)

    per_w = E // NW
    n_ch = per_w // CH
    mesh = plsc.VectorSubcoreMesh(core_axis_name="c", subcore_axis_name="s",
                                  num_cores=NC, num_subcores=NS)
    gather_fn = pl.kernel(
        functools.partial(_gather_body, per_w, n_ch),
        mesh=mesh,
        compiler_params=pltpu.CompilerParams(use_tc_tiling_on_sc=False,
                                             needs_layout_passes=False),
        out_type=jax.ShapeDtypeStruct((24, E), jnp.float32),
        scratch_types=(
            [pltpu.VMEM((n_ch, CH), jnp.int32)] * 2
            + [pltpu.VMEM((CH, ED), jnp.int32)] * (2 * NBUF)
            + [pltpu.VMEM((18, CH), jnp.float32)] * NBUF
            + [pltpu.SemaphoreType.DMA] * (3 * NBUF)
        ),
    )
    ij4 = edge_index.reshape(2, NW, n_ch, CH)
    gx = gather_fn(t_src, t_dst, ij4)

    ewt = edge_w.T
    be = 16000
    ne = E // be
    outt = pl.pallas_call(
        functools.partial(_mlp_body, float(cat_dim)),
        grid=(ne,),
        in_specs=[
            pl.BlockSpec((ED, be), lambda b: (0, b)),
            pl.BlockSpec((24, be), lambda b: (0, b)),
            pl.BlockSpec((ED, 2 * ED), lambda b: (0, 0)),
            pl.BlockSpec((ED, 8), lambda b: (0, 0)),
        ],
        out_specs=pl.BlockSpec((ED, be), lambda b: (0, b)),
        out_shape=jax.ShapeDtypeStruct((ED, E), jnp.float32),
    )(ewt, gx, cw, uvb)
    return outt.T


# speedup vs baseline: 1.1035x; 1.1035x over previous
import functools

import jax
import jax.numpy as jnp
from jax import lax
from jax.experimental import pallas as pl
from jax.experimental.pallas import tpu as pltpu
from jax.experimental.pallas import tpu_sc as plsc

NC = 2
NS = 16
NW = NC * NS
TW = 32
CH = 80


def _pack_bf16(lo_f32, hi_f32):
    lo = lax.bitcast_convert_type(lo_f32.astype(jnp.bfloat16), jnp.uint16)
    hi = lax.bitcast_convert_type(hi_f32.astype(jnp.bfloat16), jnp.uint16)
    return (lo.astype(jnp.int32)
            | lax.shift_left(hi.astype(jnp.int32), 16))


def _table_body(nf_ref, ma_ref, mb_ref, s1_ref, s2_ref, t1_ref, t2_ref):
    x = nf_ref[...]
    x2 = x * x
    sq = (jnp.dot(x, s1_ref[...], preferred_element_type=jnp.float32)
          + jnp.dot(x2, s2_ref[...], preferred_element_type=jnp.float32))
    pa = jnp.dot(x, ma_ref[...], preferred_element_type=jnp.float32)
    pb = jnp.dot(x, mb_ref[...], preferred_element_type=jnp.float32)
    t1_ref[...] = _pack_bf16(pa, sq)
    t2_ref[...] = _pack_bf16(pb, sq)


NBUF = 2


def _gather_body(per_w, n_ch, tsrc, tdst, ij, gx, ivm, jvm, *bufflat):
    ba = bufflat[0:NBUF]
    bb = bufflat[NBUF:2 * NBUF]
    bw = bufflat[2 * NBUF:3 * NBUF]
    bst = bufflat[3 * NBUF:4 * NBUF]
    sa = bufflat[4 * NBUF:5 * NBUF]
    sb = bufflat[5 * NBUF:6 * NBUF]
    sw = bufflat[6 * NBUF:7 * NBUF]
    wid = lax.axis_index("s") * NC + lax.axis_index("c")
    base = wid * per_w
    pltpu.sync_copy(ij.at[0, wid], ivm)
    pltpu.sync_copy(ij.at[1, wid], jvm)

    lanes = lax.iota(jnp.int32, 16)
    mhi = jnp.full((16,), -65536, jnp.int32)
    c0 = jnp.full((16,), 0, jnp.int32)
    c1 = jnp.full((16,), 1, jnp.int32)

    def start(c, k):
        pltpu.async_copy(tsrc.at[ivm.at[c]], ba[k], sa[k])
        pltpu.async_copy(tdst.at[jvm.at[c]], bb[k], sb[k])

    def process(c, k):
        off = pl.multiple_of(c * CH, 8)
        pltpu.make_async_copy(tsrc.at[ivm.at[c]], ba[k], sa[k]).wait()
        pltpu.make_async_copy(tdst.at[jvm.at[c]], bb[k], sb[k]).wait()

        @pl.when(c >= NBUF)
        def _():
            pltpu.make_async_copy(
                bw[k], gx.at[pl.ds(0, 18), pl.ds(base + off, CH)], sw[k]).wait()

        for rr in range(CH):
            va = ba[k][rr, :]
            vb = bb[k][rr, :]
            pa = (plsc.bitcast(lax.shift_left(va, 16), jnp.float32)
                  + plsc.bitcast(lax.shift_left(vb, 16), jnp.float32))
            plsc.store_scatter(bw[k], [lanes, jnp.full((16,), rr, jnp.int32)], pa)
            bst[k][rr, :] = (plsc.bitcast(lax.bitwise_and(va, mhi), jnp.float32)
                             + plsc.bitcast(lax.bitwise_and(vb, mhi), jnp.float32))
        for pp in range(CH // 16):
            rows = lanes + (16 * pp)
            sl = pl.ds(16 * pp, 16)
            bw[k][16, sl] = plsc.load_gather(bst[k], [rows, c0])
            bw[k][17, sl] = plsc.load_gather(bst[k], [rows, c1])
        pltpu.async_copy(bw[k], gx.at[pl.ds(0, 18), pl.ds(base + off, CH)], sw[k])

    for k in range(NBUF - 1):
        start(k, k)

    def body(c2, carry):
        cb = NBUF * c2

        for k in range(NBUF):
            c = cb + k

            @pl.when(c + NBUF - 1 < n_ch)
            def _(c=c, k=k):
                start(c + NBUF - 1, (k + NBUF - 1) % NBUF)

            @pl.when(c < n_ch)
            def _(c=c, k=k):
                process(c, k)

        return carry

    lax.fori_loop(0, (n_ch + NBUF - 1) // NBUF, body, 0)

    for k in range(NBUF):
        lastc = (n_ch - 1 - k) // NBUF * NBUF + k
        if lastc >= 0:
            pltpu.make_async_copy(
                bw[k], gx.at[pl.ds(0, 18), pl.ds(base + lastc * CH, CH)],
                sw[k]).wait()


def _mlp_body(cat_dim, e_ref, gx_ref, cw_ref, uvb_ref, o_ref):
    et = e_ref[...]
    g1t = gx_ref[0:16, :]
    srow = gx_ref[16:17, :]
    qrow = gx_ref[17:18, :]
    ct = cw_ref[:, 0:16]
    w2t = cw_ref[:, 16:32]
    scale = 1.0 / cat_dim
    se = jnp.sum(et, axis=0, keepdims=True)
    qe = jnp.sum(et * et, axis=0, keepdims=True)
    mu = (srow + se) * scale
    var = (qrow + qe) * scale - mu * mu
    inv = lax.rsqrt(var + 1e-5)
    ect = jnp.dot(ct, et, preferred_element_type=jnp.float32)
    u = uvb_ref[:, 0:1]
    v = uvb_ref[:, 1:2]
    b2c = uvb_ref[:, 2:3]
    y1 = (g1t + ect) * inv - u * (mu * inv) + v
    y1 = jnp.where(y1 > 0, y1, 0.01 * y1)
    y2 = jnp.dot(w2t, y1, preferred_element_type=jnp.float32) + b2c
    o_ref[...] = et + y2


def kernel(h0, edge_index, edge_w, ln_g, ln_b, W1, b1, W2, b2, r, basis):
    N, D, _ = h0.shape
    E = edge_index.shape[1]
    ED = edge_w.shape[1]
    cat_dim = 2 * D + ED
    nf = jnp.squeeze(h0, axis=-1)

    w1g = W1 * ln_g[:, None]
    a_m = w1g[:D]
    b_m = w1g[D:2 * D]
    c_m = w1g[2 * D:]
    u = jnp.sum(w1g, axis=0)
    v = ln_b @ W1 + b1
    zcol = jnp.zeros((D, ED - 2), jnp.float32)
    one = jnp.ones((D, 1), jnp.float32)
    zero = jnp.zeros((D, 1), jnp.float32)
    s1m = jnp.concatenate([one, zero, zcol], axis=1)
    s2m = jnp.concatenate([zero, one, zcol], axis=1)

    cw = jnp.concatenate([c_m.T, W2.T], axis=1)
    uvb = jnp.concatenate([u[:, None], v[:, None], b2[:, None],
                           jnp.zeros((ED, 5), jnp.float32)], axis=1)

    nb = 5
    bn = N // nb
    t_src, t_dst = pl.pallas_call(
        _table_body,
        grid=(nb,),
        in_specs=[
            pl.BlockSpec((bn, D), lambda b: (b, 0)),
            pl.BlockSpec((D, ED), lambda b: (0, 0)),
            pl.BlockSpec((D, ED), lambda b: (0, 0)),
            pl.BlockSpec((D, ED), lambda b: (0, 0)),
            pl.BlockSpec((D, ED), lambda b: (0, 0)),
        ],
        out_specs=[
            pl.BlockSpec((bn, ED), lambda b: (b, 0)),
            pl.BlockSpec((bn, ED), lambda b: (b, 0)),
        ],
        out_shape=[
            jax.ShapeDtypeStruct((N, ED), jnp.int32),
            jax.ShapeDtypeStruct((N, ED), jnp.int32),
        ],
    )(nf, a_m, b_m, s1m, s2m)

    per_w = E // NW
    n_ch = per_w // CH
    mesh = plsc.VectorSubcoreMesh(core_axis_name="c", subcore_axis_name="s",
                                  num_cores=NC, num_subcores=NS)
    gather_fn = pl.kernel(
        functools.partial(_gather_body, per_w, n_ch),
        mesh=mesh,
        compiler_params=pltpu.CompilerParams(use_tc_tiling_on_sc=False,
                                             needs_layout_passes=False),
        out_type=jax.ShapeDtypeStruct((24, E), jnp.float32),
        scratch_types=(
            [pltpu.VMEM((n_ch, CH), jnp.int32)] * 2
            + [pltpu.VMEM((CH, ED), jnp.int32)] * (2 * NBUF)
            + [pltpu.VMEM((18, CH), jnp.float32)] * NBUF
            + [pltpu.VMEM((CH, ED), jnp.float32)] * NBUF
            + [pltpu.SemaphoreType.DMA] * (3 * NBUF)
        ),
    )
    ij4 = edge_index.reshape(2, NW, n_ch, CH)
    gx = gather_fn(t_src, t_dst, ij4)

    ewt = edge_w.T
    be = 16000
    ne = E // be
    outt = pl.pallas_call(
        functools.partial(_mlp_body, float(cat_dim)),
        grid=(ne,),
        in_specs=[
            pl.BlockSpec((ED, be), lambda b: (0, b)),
            pl.BlockSpec((24, be), lambda b: (0, b)),
            pl.BlockSpec((ED, 2 * ED), lambda b: (0, 0)),
            pl.BlockSpec((ED, 8), lambda b: (0, 0)),
        ],
        out_specs=pl.BlockSpec((ED, be), lambda b: (0, b)),
        out_shape=jax.ShapeDtypeStruct((ED, E), jnp.float32),
    )(ewt, gx, cw, uvb)
    return outt.T


# speedup vs baseline: 1.1420x; 1.0349x over previous
import functools

import jax
import jax.numpy as jnp
from jax import lax
from jax.experimental import pallas as pl
from jax.experimental.pallas import tpu as pltpu
from jax.experimental.pallas import tpu_sc as plsc

NC = 2
NS = 16
NW = NC * NS
TW = 32
CH = 80


def _pack_bf16(lo_f32, hi_f32):
    lo = lax.bitcast_convert_type(lo_f32.astype(jnp.bfloat16), jnp.uint16)
    hi = lax.bitcast_convert_type(hi_f32.astype(jnp.bfloat16), jnp.uint16)
    return (lo.astype(jnp.int32)
            | lax.shift_left(hi.astype(jnp.int32), 16))


def _table_body(nf_ref, ma_ref, mb_ref, s1_ref, s2_ref, t1_ref, t2_ref):
    x = nf_ref[...]
    x2 = x * x
    sq = (jnp.dot(x, s1_ref[...], preferred_element_type=jnp.float32)
          + jnp.dot(x2, s2_ref[...], preferred_element_type=jnp.float32))
    pa = jnp.dot(x, ma_ref[...], preferred_element_type=jnp.float32)
    pb = jnp.dot(x, mb_ref[...], preferred_element_type=jnp.float32)
    t1_ref[...] = _pack_bf16(pa, sq)
    t2_ref[...] = _pack_bf16(pb, sq)


NBUF = 2


def _gather_body(per_w, n_ch, tsrc, tdst, ij, gx, ivm, jvm, *bufflat):
    ba = bufflat[0:NBUF]
    bb = bufflat[NBUF:2 * NBUF]
    bw = bufflat[2 * NBUF:3 * NBUF]
    bst = bufflat[3 * NBUF:4 * NBUF]
    sa = bufflat[4 * NBUF:5 * NBUF]
    sb = bufflat[5 * NBUF:6 * NBUF]
    sw = bufflat[6 * NBUF:7 * NBUF]
    wid = lax.axis_index("s") * NC + lax.axis_index("c")
    base = wid * per_w
    pltpu.sync_copy(ij.at[0, wid], ivm)
    pltpu.sync_copy(ij.at[1, wid], jvm)

    lanes = lax.iota(jnp.int32, 16)
    mhi = jnp.full((16,), -65536, jnp.int32)
    c0 = jnp.full((16,), 0, jnp.int32)
    c1 = jnp.full((16,), 1, jnp.int32)

    def start(c, k):
        pltpu.async_copy(tsrc.at[ivm.at[c]], ba[k], sa[k])
        pltpu.async_copy(tdst.at[jvm.at[c]], bb[k], sb[k])

    def process(c, k):
        off = pl.multiple_of(c * CH, 8)
        pltpu.make_async_copy(tsrc.at[ivm.at[c]], ba[k], sa[k]).wait()
        pltpu.make_async_copy(tdst.at[jvm.at[c]], bb[k], sb[k]).wait()

        @pl.when(c >= NBUF)
        def _():
            pltpu.make_async_copy(
                bw[k], gx.at[pl.ds(0, 18), pl.ds(base + off, CH)], sw[k]).wait()

        for rr in range(CH):
            va = ba[k][rr, :]
            vb = bb[k][rr, :]
            pa = (plsc.bitcast(lax.shift_left(va, 16), jnp.float32)
                  + plsc.bitcast(lax.shift_left(vb, 16), jnp.float32))
            plsc.store_scatter(bw[k], [lanes, jnp.full((16,), rr, jnp.int32)], pa)
            bst[k][rr, :] = (plsc.bitcast(lax.bitwise_and(va, mhi), jnp.float32)
                             + plsc.bitcast(lax.bitwise_and(vb, mhi), jnp.float32))
        for pp in range(CH // 16):
            rows = lanes + (16 * pp)
            sl = pl.ds(16 * pp, 16)
            bw[k][16, sl] = plsc.load_gather(bst[k], [rows, c0])
            bw[k][17, sl] = plsc.load_gather(bst[k], [rows, c1])
        pltpu.async_copy(bw[k], gx.at[pl.ds(0, 18), pl.ds(base + off, CH)], sw[k])

    for k in range(NBUF - 1):
        start(k, k)

    def body(c2, carry):
        cb = NBUF * c2

        for k in range(NBUF):
            c = cb + k

            @pl.when(c + NBUF - 1 < n_ch)
            def _(c=c, k=k):
                start(c + NBUF - 1, (k + NBUF - 1) % NBUF)

            @pl.when(c < n_ch)
            def _(c=c, k=k):
                process(c, k)

        return carry

    lax.fori_loop(0, (n_ch + NBUF - 1) // NBUF, body, 0)

    for k in range(NBUF):
        lastc = (n_ch - 1 - k) // NBUF * NBUF + k
        if lastc >= 0:
            pltpu.make_async_copy(
                bw[k], gx.at[pl.ds(0, 18), pl.ds(base + lastc * CH, CH)],
                sw[k]).wait()


def _mlp_body(cat_dim, e_ref, gx_ref, cw_ref, uvb_ref, o_ref):
    et = e_ref[...]
    g1t = gx_ref[0:16, :]
    srow = gx_ref[16:17, :]
    qrow = gx_ref[17:18, :]
    ct = cw_ref[:, 0:16]
    w2t = cw_ref[:, 16:32]
    scale = 1.0 / cat_dim
    se = jnp.sum(et, axis=0, keepdims=True)
    qe = jnp.sum(et * et, axis=0, keepdims=True)
    mu = (srow + se) * scale
    var = (qrow + qe) * scale - mu * mu
    inv = lax.rsqrt(var + 1e-5)
    ect = jnp.dot(ct, et, preferred_element_type=jnp.float32)
    u = uvb_ref[:, 0:1]
    v = uvb_ref[:, 1:2]
    b2c = uvb_ref[:, 2:3]
    y1 = (g1t + ect) * inv - u * (mu * inv) + v
    y1 = jnp.where(y1 > 0, y1, 0.01 * y1)
    y2 = jnp.dot(w2t, y1, preferred_element_type=jnp.float32) + b2c
    o_ref[...] = et + y2


def kernel(h0, edge_index, edge_w, ln_g, ln_b, W1, b1, W2, b2, r, basis):
    N, D, _ = h0.shape
    E = edge_index.shape[1]
    ED = edge_w.shape[1]
    cat_dim = 2 * D + ED
    nf = jnp.squeeze(h0, axis=-1)

    w1g = W1 * ln_g[:, None]
    a_m = w1g[:D]
    b_m = w1g[D:2 * D]
    c_m = w1g[2 * D:]
    u = jnp.sum(w1g, axis=0)
    v = ln_b @ W1 + b1
    zcol = jnp.zeros((D, ED - 2), jnp.float32)
    one = jnp.ones((D, 1), jnp.float32)
    zero = jnp.zeros((D, 1), jnp.float32)
    s1m = jnp.concatenate([one, zero, zcol], axis=1)
    s2m = jnp.concatenate([zero, one, zcol], axis=1)

    cw = jnp.concatenate([c_m.T, W2.T], axis=1)
    uvb = jnp.concatenate([u[:, None], v[:, None], b2[:, None],
                           jnp.zeros((ED, 5), jnp.float32)], axis=1)

    nb = 5
    bn = N // nb
    t_src, t_dst = pl.pallas_call(
        _table_body,
        grid=(nb,),
        in_specs=[
            pl.BlockSpec((bn, D), lambda b: (b, 0)),
            pl.BlockSpec((D, ED), lambda b: (0, 0)),
            pl.BlockSpec((D, ED), lambda b: (0, 0)),
            pl.BlockSpec((D, ED), lambda b: (0, 0)),
            pl.BlockSpec((D, ED), lambda b: (0, 0)),
        ],
        out_specs=[
            pl.BlockSpec((bn, ED), lambda b: (b, 0)),
            pl.BlockSpec((bn, ED), lambda b: (b, 0)),
        ],
        out_shape=[
            jax.ShapeDtypeStruct((N, ED), jnp.int32),
            jax.ShapeDtypeStruct((N, ED), jnp.int32),
        ],
    )(nf, a_m, b_m, s1m, s2m)

    per_w = E // NW
    n_ch = per_w // CH
    mesh = plsc.VectorSubcoreMesh(core_axis_name="c", subcore_axis_name="s",
                                  num_cores=NC, num_subcores=NS)
    gather_fn = pl.kernel(
        functools.partial(_gather_body, per_w, n_ch),
        mesh=mesh,
        compiler_params=pltpu.CompilerParams(use_tc_tiling_on_sc=False,
                                             needs_layout_passes=False),
        out_type=jax.ShapeDtypeStruct((18, E), jnp.float32),
        scratch_types=(
            [pltpu.VMEM((n_ch, CH), jnp.int32)] * 2
            + [pltpu.VMEM((CH, ED), jnp.int32)] * (2 * NBUF)
            + [pltpu.VMEM((18, CH), jnp.float32)] * NBUF
            + [pltpu.VMEM((CH, ED), jnp.float32)] * NBUF
            + [pltpu.SemaphoreType.DMA] * (3 * NBUF)
        ),
    )
    ij4 = edge_index.reshape(2, NW, n_ch, CH)
    gx = gather_fn(t_src, t_dst, ij4)

    ewt = edge_w.T
    be = 32000
    ne = E // be
    outt = pl.pallas_call(
        functools.partial(_mlp_body, float(cat_dim)),
        grid=(ne,),
        in_specs=[
            pl.BlockSpec((ED, be), lambda b: (0, b)),
            pl.BlockSpec((18, be), lambda b: (0, b)),
            pl.BlockSpec((ED, 2 * ED), lambda b: (0, 0)),
            pl.BlockSpec((ED, 8), lambda b: (0, 0)),
        ],
        out_specs=pl.BlockSpec((ED, be), lambda b: (0, b)),
        out_shape=jax.ShapeDtypeStruct((ED, E), jnp.float32),
    )(ewt, gx, cw, uvb)
    return outt.T


# speedup vs baseline: 1.1423x; 1.0003x over previous
import functools

import jax
import jax.numpy as jnp
from jax import lax
from jax.experimental import pallas as pl
from jax.experimental.pallas import tpu as pltpu
from jax.experimental.pallas import tpu_sc as plsc

NC = 2
NS = 16
NW = NC * NS
CH = 80


def _pack_bf16(lo_f32, hi_f32):
    lo = lax.bitcast_convert_type(lo_f32.astype(jnp.bfloat16), jnp.uint16)
    hi = lax.bitcast_convert_type(hi_f32.astype(jnp.bfloat16), jnp.uint16)
    return (lo.astype(jnp.int32)
            | lax.shift_left(hi.astype(jnp.int32), 16))


def _table_body(nf_ref, ma_ref, mb_ref, s1_ref, s2_ref, t1_ref, t2_ref):
    x = nf_ref[...]
    x2 = x * x
    sq = (jnp.dot(x, s1_ref[...], preferred_element_type=jnp.float32)
          + jnp.dot(x2, s2_ref[...], preferred_element_type=jnp.float32))
    pa = jnp.dot(x, ma_ref[...], preferred_element_type=jnp.float32)
    pb = jnp.dot(x, mb_ref[...], preferred_element_type=jnp.float32)
    t1_ref[...] = _pack_bf16(pa, sq)
    t2_ref[...] = _pack_bf16(pb, sq)


NBUF = 2


def _gather_body(per_w, n_ch, tsrc, tdst, ij, gx, ivm, jvm, *bufflat):
    ba = bufflat[0:NBUF]
    bb = bufflat[NBUF:2 * NBUF]
    bw = bufflat[2 * NBUF:3 * NBUF]
    bst = bufflat[3 * NBUF:4 * NBUF]
    sa = bufflat[4 * NBUF:5 * NBUF]
    sb = bufflat[5 * NBUF:6 * NBUF]
    sw = bufflat[6 * NBUF:7 * NBUF]
    wid = lax.axis_index("s") * NC + lax.axis_index("c")
    base = wid * per_w
    pltpu.sync_copy(ij.at[0, wid], ivm)
    pltpu.sync_copy(ij.at[1, wid], jvm)

    lanes = lax.iota(jnp.int32, 16)
    mhi = jnp.full((16,), -65536, jnp.int32)
    c0 = jnp.full((16,), 0, jnp.int32)
    c1 = jnp.full((16,), 1, jnp.int32)

    def start(c, k):
        pltpu.async_copy(tsrc.at[ivm.at[c]], ba[k], sa[k])
        pltpu.async_copy(tdst.at[jvm.at[c]], bb[k], sb[k])

    def process(c, k):
        off = pl.multiple_of(c * CH, 8)
        pltpu.make_async_copy(tsrc.at[ivm.at[c]], ba[k], sa[k]).wait()
        pltpu.make_async_copy(tdst.at[jvm.at[c]], bb[k], sb[k]).wait()

        @pl.when(c >= NBUF)
        def _():
            pltpu.make_async_copy(
                bw[k], gx.at[pl.ds(0, 18), pl.ds(base + off, CH)], sw[k]).wait()

        for rr in range(CH):
            va = ba[k][rr, :]
            vb = bb[k][rr, :]
            pa = (plsc.bitcast(lax.shift_left(va, 16), jnp.float32)
                  + plsc.bitcast(lax.shift_left(vb, 16), jnp.float32))
            plsc.store_scatter(bw[k], [lanes, jnp.full((16,), rr, jnp.int32)], pa)
            bst[k][rr, :] = (plsc.bitcast(lax.bitwise_and(va, mhi), jnp.float32)
                             + plsc.bitcast(lax.bitwise_and(vb, mhi), jnp.float32))
        for pp in range(CH // 16):
            rows = lanes + (16 * pp)
            sl = pl.ds(16 * pp, 16)
            bw[k][16, sl] = plsc.load_gather(bst[k], [rows, c0])
            bw[k][17, sl] = plsc.load_gather(bst[k], [rows, c1])
        pltpu.async_copy(bw[k], gx.at[pl.ds(0, 18), pl.ds(base + off, CH)], sw[k])

    for k in range(NBUF - 1):
        start(k, k)

    def body(c2, carry):
        cb = NBUF * c2

        for k in range(NBUF):
            c = cb + k

            @pl.when(c + NBUF - 1 < n_ch)
            def _(c=c, k=k):
                start(c + NBUF - 1, (k + NBUF - 1) % NBUF)

            @pl.when(c < n_ch)
            def _(c=c, k=k):
                process(c, k)

        return carry

    lax.fori_loop(0, (n_ch + NBUF - 1) // NBUF, body, 0)

    for k in range(NBUF):
        lastc = (n_ch - 1 - k) // NBUF * NBUF + k
        if lastc >= 0:
            pltpu.make_async_copy(
                bw[k], gx.at[pl.ds(0, 18), pl.ds(base + lastc * CH, CH)],
                sw[k]).wait()


def _mlp_body(cat_dim, e_ref, gx_ref, cw_ref, uvb_ref, o_ref):
    et = e_ref[...]
    g1t = gx_ref[0:16, :]
    srow = gx_ref[16:17, :]
    qrow = gx_ref[17:18, :]
    ct = cw_ref[:, 0:16]
    w2t = cw_ref[:, 16:32]
    scale = 1.0 / cat_dim
    se = jnp.sum(et, axis=0, keepdims=True)
    qe = jnp.sum(et * et, axis=0, keepdims=True)
    mu = (srow + se) * scale
    var = (qrow + qe) * scale - mu * mu
    inv = lax.rsqrt(var + 1e-5)
    ect = jnp.dot(ct, et, preferred_element_type=jnp.float32)
    u = uvb_ref[:, 0:1]
    v = uvb_ref[:, 1:2]
    b2c = uvb_ref[:, 2:3]
    y1 = (g1t + ect) * inv - u * (mu * inv) + v
    y1 = jnp.where(y1 > 0, y1, 0.01 * y1)
    y2 = jnp.dot(w2t, y1, preferred_element_type=jnp.float32) + b2c
    o_ref[...] = et + y2


def kernel(h0, edge_index, edge_w, ln_g, ln_b, W1, b1, W2, b2, r, basis):
    N, D, _ = h0.shape
    E = edge_index.shape[1]
    ED = edge_w.shape[1]
    cat_dim = 2 * D + ED
    nf = jnp.squeeze(h0, axis=-1)

    w1g = W1 * ln_g[:, None]
    a_m = w1g[:D]
    b_m = w1g[D:2 * D]
    c_m = w1g[2 * D:]
    u = jnp.sum(w1g, axis=0)
    v = ln_b @ W1 + b1
    zcol = jnp.zeros((D, ED - 2), jnp.float32)
    one = jnp.ones((D, 1), jnp.float32)
    zero = jnp.zeros((D, 1), jnp.float32)
    s1m = jnp.concatenate([one, zero, zcol], axis=1)
    s2m = jnp.concatenate([zero, one, zcol], axis=1)

    cw = jnp.concatenate([c_m.T, W2.T], axis=1)
    uvb = jnp.concatenate([u[:, None], v[:, None], b2[:, None],
                           jnp.zeros((ED, 5), jnp.float32)], axis=1)

    nb = 5
    bn = N // nb
    t_src, t_dst = pl.pallas_call(
        _table_body,
        grid=(nb,),
        in_specs=[
            pl.BlockSpec((bn, D), lambda b: (b, 0)),
            pl.BlockSpec((D, ED), lambda b: (0, 0)),
            pl.BlockSpec((D, ED), lambda b: (0, 0)),
            pl.BlockSpec((D, ED), lambda b: (0, 0)),
            pl.BlockSpec((D, ED), lambda b: (0, 0)),
        ],
        out_specs=[
            pl.BlockSpec((bn, ED), lambda b: (b, 0)),
            pl.BlockSpec((bn, ED), lambda b: (b, 0)),
        ],
        out_shape=[
            jax.ShapeDtypeStruct((N, ED), jnp.int32),
            jax.ShapeDtypeStruct((N, ED), jnp.int32),
        ],
    )(nf, a_m, b_m, s1m, s2m)

    per_w = E // NW
    n_ch = per_w // CH
    mesh = plsc.VectorSubcoreMesh(core_axis_name="c", subcore_axis_name="s",
                                  num_cores=NC, num_subcores=NS)
    gather_fn = pl.kernel(
        functools.partial(_gather_body, per_w, n_ch),
        mesh=mesh,
        compiler_params=pltpu.CompilerParams(use_tc_tiling_on_sc=False,
                                             needs_layout_passes=False),
        out_type=jax.ShapeDtypeStruct((18, E), jnp.float32),
        scratch_types=(
            [pltpu.VMEM((n_ch, CH), jnp.int32)] * 2
            + [pltpu.VMEM((CH, ED), jnp.int32)] * (2 * NBUF)
            + [pltpu.VMEM((18, CH), jnp.float32)] * NBUF
            + [pltpu.VMEM((CH, ED), jnp.float32)] * NBUF
            + [pltpu.SemaphoreType.DMA] * (3 * NBUF)
        ),
    )
    ij4 = edge_index.reshape(2, NW, n_ch, CH)
    gx = gather_fn(t_src, t_dst, ij4)

    ewt = edge_w.T
    be = 32000
    ne = E // be
    outt = pl.pallas_call(
        functools.partial(_mlp_body, float(cat_dim)),
        grid=(ne,),
        in_specs=[
            pl.BlockSpec((ED, be), lambda b: (0, b)),
            pl.BlockSpec((18, be), lambda b: (0, b)),
            pl.BlockSpec((ED, 2 * ED), lambda b: (0, 0)),
            pl.BlockSpec((ED, 8), lambda b: (0, 0)),
        ],
        out_specs=pl.BlockSpec((ED, be), lambda b: (0, b)),
        out_shape=jax.ShapeDtypeStruct((ED, E), jnp.float32),
    )(ewt, gx, cw, uvb)
    return outt.T
